# baseline (speedup 1.0000x reference)
import jax
import jax.numpy as jnp
from jax import lax
from jax.experimental import pallas as pl
from jax.experimental.pallas import tpu as pltpu
from jax.experimental.pallas import tpu_sc as plsc

B, S, V, D = 4096, 200, 64, 64
NC, NS, L = 2, 16, 16
NW = NC * NS
RPW = B // NW
TPW = RPW * S
CHUNK = 128
NCHUNK = TPW // CHUNK
NVREG = (S + L - 1) // L

THRESHOLDS = (1, 2, 7, 20, 54, 148, 403, 1096, 2980, 8103,
              22026, 59874, 162754, 442413)


def _sc_body(ts_hbm, te_hbm, out_hbm, ts_v, idx_v, rows0_v, rows1_v,
             sem0, sem1):
    wid = lax.axis_index("s") * NC + lax.axis_index("c")
    base = wid * RPW
    tok0 = base * S

    pltpu.sync_copy(ts_hbm.at[pl.ds(tok0, TPW)], ts_v)

    def compute_row(r, carry):
        rb = r * S
        cur = jnp.max(lax.div(ts_v[pl.ds(rb + S - L, L)], jnp.int32(3600)))
        for j in range(NVREG):
            off = j * L if j < NVREG - 1 else S - L
            d = cur - lax.div(ts_v[pl.ds(rb + off, L)], jnp.int32(3600))
            acc = (d >= THRESHOLDS[0]).astype(jnp.int32)
            for t in THRESHOLDS[1:]:
                acc = acc + (d >= t).astype(jnp.int32)
            idx_v[pl.ds(rb + off, L)] = acc
        return carry

    lax.fori_loop(0, RPW, compute_row, 0, unroll=False)

    pltpu.async_copy(te_hbm.at[idx_v.at[pl.ds(0, CHUNK)]], rows0_v, sem0)

    def gather_chunk(g, carry):
        slot = lax.rem(g, 2)

        @pl.when(jnp.logical_and(slot == 0, g + 1 < NCHUNK))
        def _():
            pltpu.async_copy(
                te_hbm.at[idx_v.at[pl.ds((g + 1) * CHUNK, CHUNK)]],
                rows1_v, sem1)

        @pl.when(jnp.logical_and(slot == 1, g + 1 < NCHUNK))
        def _():
            pltpu.async_copy(
                te_hbm.at[idx_v.at[pl.ds((g + 1) * CHUNK, CHUNK)]],
                rows0_v, sem0)

        @pl.when(slot == 0)
        def _():
            pltpu.make_async_copy(
                te_hbm.at[idx_v.at[pl.ds(g * CHUNK, CHUNK)]],
                rows0_v, sem0).wait()
            pltpu.sync_copy(rows0_v, out_hbm.at[pl.ds(tok0 + g * CHUNK,
                                                      CHUNK)])

        @pl.when(slot == 1)
        def _():
            pltpu.make_async_copy(
                te_hbm.at[idx_v.at[pl.ds(g * CHUNK, CHUNK)]],
                rows1_v, sem1).wait()
            pltpu.sync_copy(rows1_v, out_hbm.at[pl.ds(tok0 + g * CHUNK,
                                                      CHUNK)])

        return carry

    lax.fori_loop(0, NCHUNK, gather_chunk, 0, unroll=False)


@jax.jit
def _time_embedding(timestamps, te):
    mesh = plsc.VectorSubcoreMesh(core_axis_name="c", subcore_axis_name="s")
    f = pl.kernel(
        _sc_body,
        out_type=jax.ShapeDtypeStruct((B * S, D), jnp.float32),
        mesh=mesh,
        scratch_types=[
            pltpu.VMEM((TPW,), jnp.int32),
            pltpu.VMEM((TPW,), jnp.int32),
            pltpu.VMEM((CHUNK, D), jnp.float32),
            pltpu.VMEM((CHUNK, D), jnp.float32),
            pltpu.SemaphoreType.DMA,
            pltpu.SemaphoreType.DMA,
        ],
        compiler_params=pltpu.CompilerParams(use_tc_tiling_on_sc=False,
                                             needs_layout_passes=False),
    )
    return f(timestamps.reshape(B * S), te).reshape(B, S, D)


def kernel(timestamps, te):
    return _time_embedding(timestamps, te)


# speedup vs baseline: 115.8825x; 115.8825x over previous
import jax
import jax.numpy as jnp
from jax import lax
from jax.experimental import pallas as pl
from jax.experimental.pallas import tpu as pltpu
from jax.experimental.pallas import tpu_sc as plsc

B, S, V, D = 4096, 200, 64, 64
NC, NS, L = 2, 16, 16
NW = NC * NS
BPW = B // NW
TPW = BPW * S
NVREG = (S + L - 1) // L
PAD = 65
SBLK = 2
NBLK = S // SBLK
BLKF = SBLK * D * BPW

THRESHOLDS = (1, 2, 7, 20, 54, 148, 403, 1096, 2980, 8103,
              22026, 59874, 162754, 442413)


def _sc_body(ts_hbm, te_hbm, out_hbm, ts_v, idx_v, te_v, tep_v,
             buf0_v, buf1_v, wsem0, wsem1):
    wid = lax.axis_index("s") * NC + lax.axis_index("c")
    tok0 = wid * TPW
    b0 = wid * BPW

    pltpu.sync_copy(ts_hbm.at[pl.ds(tok0, TPW)], ts_v)
    pltpu.sync_copy(te_hbm, te_v)

    iota = lax.iota(jnp.int32, L)

    def pad_row(v, carry):
        dst0 = v * PAD
        src0 = v * D
        for j in range(D // L):
            val = te_v[pl.ds(src0 + j * L, L)]
            plsc.store_scatter(tep_v, [dst0 + j * L + iota], val)
        return carry

    lax.fori_loop(0, V, pad_row, 0, unroll=False)

    @plsc.parallel_loop(0, BPW, step=1, unroll=1)
    def compute_row(r):
        rb = r * S
        cur_raw = jnp.max(ts_v[pl.ds(rb + S - L, L)])
        c3600 = lax.div(cur_raw, jnp.int32(3600)) * jnp.int32(3600)
        cuts = [c3600 - jnp.int32(3600 * (s - 1)) for s in THRESHOLDS]
        for j in range(NVREG):
            off = j * L if j < NVREG - 1 else S - L
            v = ts_v[pl.ds(rb + off, L)]
            acc = (v < cuts[0]).astype(jnp.int32)
            for c in cuts[1:]:
                acc = acc + (v < c).astype(jnp.int32)
            idx_v[pl.ds(rb + off, L)] = acc

    bufs = (buf0_v, buf1_v)
    sems = (wsem0, wsem1)
    NG = BPW // L

    bgbase = [iota * jnp.int32(S) + jnp.int32(bg * L * S) for bg in range(NG)]

    def fill_block(blk, buf):
        for ss in range(SBLK):
            s = blk * SBLK + ss
            i65 = []
            for bg in range(NG):
                iv = plsc.load_gather(idx_v, [bgbase[bg] + s])
                i65.append(iv * jnp.int32(PAD))

            @plsc.parallel_loop(0, D, step=1, unroll=8)
            def _cloop(c):
                row = ss * D + c
                for bg in range(NG):
                    val = plsc.load_gather(tep_v, [i65[bg] + c])
                    buf[row, pl.ds(bg * L, L)] = val

    def outer(o, carry):
        for b in range(2):
            blk = o * 2 + b

            @pl.when(o > 0)
            def _():
                pltpu.make_async_copy(
                    bufs[b],
                    out_hbm.at[pl.ds((blk - 2) * SBLK * D, SBLK * D),
                               pl.ds(b0, BPW)],
                    sems[b]).wait()

            fill_block(blk, bufs[b])
            pltpu.async_copy(
                bufs[b],
                out_hbm.at[pl.ds(blk * SBLK * D, SBLK * D),
                           pl.ds(b0, BPW)],
                sems[b])
        return carry

    lax.fori_loop(0, NBLK // 2, outer, 0, unroll=False)

    for b in range(2):
        last_blk = NBLK - 2 + b
        pltpu.make_async_copy(
            bufs[b],
            out_hbm.at[pl.ds(last_blk * SBLK * D, SBLK * D),
                       pl.ds(b0, BPW)],
            sems[b]).wait()


@jax.jit
def _time_embedding(timestamps, te):
    mesh = plsc.VectorSubcoreMesh(core_axis_name="c", subcore_axis_name="s")
    f = pl.kernel(
        _sc_body,
        out_type=jax.ShapeDtypeStruct((S * D, B), jnp.float32),
        mesh=mesh,
        scratch_types=[
            pltpu.VMEM((TPW,), jnp.int32),
            pltpu.VMEM((TPW,), jnp.int32),
            pltpu.VMEM((V * D,), jnp.float32),
            pltpu.VMEM((V * PAD,), jnp.float32),
            pltpu.VMEM((SBLK * D, BPW), jnp.float32),
            pltpu.VMEM((SBLK * D, BPW), jnp.float32),
            pltpu.SemaphoreType.DMA,
            pltpu.SemaphoreType.DMA,
        ],
        compiler_params=pltpu.CompilerParams(needs_layout_passes=False),
    )
    out = f(timestamps.reshape(B * S), te.reshape(V * D))
    return jnp.transpose(out.reshape(S, D, B), (2, 0, 1))


def kernel(timestamps, te):
    return _time_embedding(timestamps, te)
